# Initial kernel scaffold; baseline (speedup 1.0000x reference)
import jax
import jax.numpy as jnp
from jax.experimental import pallas as pl


def kernel(sentence, table, Wi, Wh, b, fcw, fcb):
    raise NotImplementedError("write your pallas kernel here")


# speedup vs baseline: 1.1595x; 1.1595x over previous
import jax
import jax.numpy as jnp
from jax.experimental import pallas as pl
from jax.experimental.pallas import tpu as pltpu


def _lstm_lm_kernel(S, E, Hp, G, V):
    def body(sent_ref, table_hbm, wi_ref, wh_ref, b_ref, fcw_hbm, fcb_ref,
             out_ref, emb_ref, fcw_ref, gates_ref, hid_ref, emb_sem, fcw_sem):
        fcw_copy = pltpu.make_async_copy(fcw_hbm, fcw_ref, fcw_sem)
        fcw_copy.start()

        copies = []
        for t in range(S):
            base = pl.multiple_of((sent_ref[t] >> 3) << 3, 8)
            cp = pltpu.make_async_copy(
                table_hbm.at[pl.ds(base, 8), :], emb_ref.at[t], emb_sem)
            cp.start()
            copies.append(cp)
        for cp in copies:
            cp.wait()

        iota8 = jax.lax.broadcasted_iota(jnp.int32, (8, E), 0)
        rows = []
        for t in range(S):
            mask = (iota8 == (sent_ref[t] & 7)).astype(jnp.float32)
            rows.append(jnp.sum(emb_ref[t] * mask, axis=0, keepdims=True))
        emb = jnp.concatenate(rows, axis=0)

        gates_ref[...] = jnp.dot(
            emb, wi_ref[...], preferred_element_type=jnp.float32) + b_ref[...]

        wh = wh_ref[...]
        h = jnp.zeros((1, Hp), jnp.float32)
        c = jnp.zeros((1, Hp), jnp.float32)
        for t in range(S):
            gates = gates_ref[t:t + 1, :] + jnp.dot(
                h, wh, preferred_element_type=jnp.float32)
            sg = jax.nn.sigmoid(gates)
            i_g = sg[:, 0 * Hp:1 * Hp]
            f_g = sg[:, 1 * Hp:2 * Hp]
            g_g = 2.0 * sg[:, 2 * Hp:3 * Hp] - 1.0
            o_g = sg[:, 3 * Hp:4 * Hp]
            c = f_g * c + i_g * g_g
            h = o_g * jnp.tanh(c)
            hid_ref[t:t + 1, :] = h

        fcw_copy.wait()
        logits = jnp.dot(hid_ref[...], fcw_ref[...],
                         preferred_element_type=jnp.float32) + fcb_ref[...]
        act = jnp.maximum(logits, 0.0)
        m = jnp.max(act, axis=1, keepdims=True)
        lse = jnp.log(jnp.sum(jnp.exp(act - m), axis=1, keepdims=True)) + m
        out_ref[...] = act - lse

    return body


def kernel(sentence, table, Wi, Wh, b, fcw, fcb):
    sent = sentence.reshape(-1).astype(jnp.int32)
    S = sent.shape[0]
    V, E = table.shape
    Hp, G = Wh.shape
    Vout = fcw.shape[1]

    def full(shape):
        return pl.BlockSpec(shape, lambda i, s: (0,) * len(shape))

    grid_spec = pltpu.PrefetchScalarGridSpec(
        num_scalar_prefetch=1,
        grid=(1,),
        in_specs=[
            pl.BlockSpec(memory_space=pl.ANY),
            full((E, G)),
            full((Hp, G)),
            full((1, G)),
            pl.BlockSpec(memory_space=pl.ANY),
            full((1, Vout)),
        ],
        out_specs=full((S, Vout)),
        scratch_shapes=[
            pltpu.VMEM((S, 8, E), jnp.float32),
            pltpu.VMEM((Hp, Vout), jnp.float32),
            pltpu.VMEM((S, G), jnp.float32),
            pltpu.VMEM((S, Hp), jnp.float32),
            pltpu.SemaphoreType.DMA,
            pltpu.SemaphoreType.DMA,
        ],
    )
    return pl.pallas_call(
        _lstm_lm_kernel(S, E, Hp, G, Vout),
        out_shape=jax.ShapeDtypeStruct((S, Vout), jnp.float32),
        grid_spec=grid_spec,
        compiler_params=pltpu.CompilerParams(
            dimension_semantics=("arbitrary",),
            vmem_limit_bytes=100 * 1024 * 1024,
        ),
    )(sent, table, Wi, Wh, b, fcw, fcb)
